# Initial kernel scaffold; baseline (speedup 1.0000x reference)
import jax
import jax.numpy as jnp
from jax.experimental import pallas as pl


def kernel(log_probs, input_lens, labels, A_scores):
    raise NotImplementedError("write your pallas kernel here")


# speedup vs baseline: 9.8393x; 9.8393x over previous
import jax
import jax.numpy as jnp
from jax.experimental import pallas as pl
from jax.experimental.pallas import tpu as pltpu

B, T, L, C = 16, 4096, 31, 32
ROWS = B * T


def _crf_body(x_ref, lab_ref, nxt_ref, arest_ref, astart_ref, lab0_ref,
              last_ref, out_ref):
    pid = pl.program_id(0)
    nprog = pl.num_programs(0)

    x = x_ref[...]
    lab = lab_ref[...]
    nxt = nxt_ref[...]
    R = x.shape[0]

    iota = jax.lax.broadcasted_iota(jnp.int32, (R, C), 1)
    onehot_lab = (lab == iota)
    onehot_nxt = (nxt == iota)

    m = jnp.max(x, axis=1, keepdims=True)
    lse32 = m[:, 0] + jnp.log(jnp.sum(jnp.exp(x - m), axis=1))
    emis = jnp.sum(jnp.where(onehot_lab, x, 0.0), axis=1)

    arest = arest_ref[...]
    am = jnp.max(arest, axis=1, keepdims=True)
    rest = arest - (am + jnp.log(jnp.sum(jnp.exp(arest - am), axis=1,
                                         keepdims=True)))
    rows = jnp.dot(onehot_lab.astype(jnp.float32), rest,
                   preferred_element_type=jnp.float32)
    trans = jnp.sum(jnp.where(onehot_nxt, rows, 0.0))

    part = jnp.sum(emis) - jnp.sum(lse32) + trans

    @pl.when(pid == 0)
    def _init():
        astart = astart_ref[...]
        sm = jnp.max(astart)
        s_lse = sm + jnp.log(jnp.sum(jnp.exp(astart - sm)))
        log_start = astart - s_lse
        lab0 = lab0_ref[...]
        i0 = jax.lax.broadcasted_iota(jnp.int32, (B, C), 1)
        start_sum = jnp.sum(jnp.where(lab0 == i0,
                                      jnp.broadcast_to(log_start, (B, C)),
                                      0.0))
        xl = last_ref[...]
        ml = jnp.max(xl, axis=1, keepdims=True)
        e = jnp.exp(xl - ml)
        s32 = jnp.sum(e, axis=1)
        s31 = s32 - e[:, C - 1]
        corr = jnp.sum(jnp.log(s32) - jnp.log(s31))
        out_ref[...] = jnp.reshape(start_sum + corr, (1, 1))

    out_ref[...] += jnp.reshape(part, (1, 1))

    @pl.when(pid == nprog - 1)
    def _fin():
        out_ref[...] = out_ref[...] / float(ROWS)


def kernel(log_probs, input_lens, labels, A_scores):
    del input_lens
    x2d = log_probs.reshape(ROWS, C)
    lab2d = labels.reshape(ROWS, 1)
    nxt = jnp.concatenate(
        [labels[:, 1:], jnp.full((B, 1), L, dtype=labels.dtype)], axis=1
    ).reshape(ROWS, 1)
    arest = jnp.concatenate(
        [A_scores[L:].reshape(L, C), jnp.zeros((1, C), jnp.float32)], axis=0)
    astart = jnp.concatenate(
        [A_scores[:L], jnp.full((1,), -1e30, jnp.float32)]).reshape(1, C)
    lab0 = labels[:, :1]
    last = log_probs[:, -1, :]

    G = 16
    R = ROWS // G

    out = pl.pallas_call(
        _crf_body,
        grid=(G,),
        in_specs=[
            pl.BlockSpec((R, C), lambda i: (i, 0)),
            pl.BlockSpec((R, 1), lambda i: (i, 0)),
            pl.BlockSpec((R, 1), lambda i: (i, 0)),
            pl.BlockSpec((C, C), lambda i: (0, 0)),
            pl.BlockSpec((1, C), lambda i: (0, 0)),
            pl.BlockSpec((B, 1), lambda i: (0, 0)),
            pl.BlockSpec((B, C), lambda i: (0, 0)),
        ],
        out_specs=pl.BlockSpec((1, 1), lambda i: (0, 0)),
        out_shape=jax.ShapeDtypeStruct((1, 1), jnp.float32),
    )(x2d, lab2d, nxt, arest, astart, lab0, last)
    return out[0, 0]


# speedup vs baseline: 10.7051x; 1.0880x over previous
import functools

import jax
import jax.numpy as jnp
from jax import lax
from jax.experimental import pallas as pl
from jax.experimental.pallas import tpu as pltpu
from jax.experimental.pallas import tpu_sc as plsc

B, T, L, C = 16, 4096, 31, 32
ROWS = B * T
NLANE = ROWS * C // 128
GRID = 16
NBLK = NLANE // GRID

NW = 32
WCHUNK = ROWS // NW
KV = WCHUNK // 16
NROWIDX = 16


def _tc_body(x_ref, arestT_ref, astart_ref, last_ref,
             out_ref, tab_ref, astartn_ref):
    pid = pl.program_id(0)

    x = x_ref[...]
    e = jnp.exp(x).astype(jnp.bfloat16)
    ii = lax.broadcasted_iota(jnp.int32, (128, 128), 0) // 32
    jj = lax.broadcasted_iota(jnp.int32, (128, 128), 1) // 32
    p = (ii == jj).astype(jnp.bfloat16)
    s = jnp.dot(e, p, preferred_element_type=jnp.float32)
    part = -jnp.sum(jnp.log(s)) / 32.0

    @pl.when(pid == 0)
    def _once():
        at = arestT_ref[...]
        m0 = jnp.max(at, axis=0, keepdims=True)
        rowlse = m0 + jnp.log(jnp.sum(jnp.exp(at - m0), axis=0, keepdims=True))
        tab_ref[...] = at - rowlse

        astart = astart_ref[...]
        sm = jnp.max(astart)
        s_lse = sm + jnp.log(jnp.sum(jnp.exp(astart - sm)))
        astartn_ref[...] = astart - s_lse

        xl = last_ref[...]
        ml = jnp.max(xl, axis=1, keepdims=True)
        el = jnp.exp(xl - ml)
        s32 = jnp.sum(el, axis=1)
        s31 = s32 - el[:, C - 1]
        corr = jnp.sum(jnp.log(s32) - jnp.log(s31))
        out_ref[...] = jnp.reshape(corr, (1, 1))

    out_ref[...] += jnp.reshape(part, (1, 1))


def _tc_call(x3, arestT, astart, last):
    return pl.pallas_call(
        _tc_body,
        grid=(GRID,),
        in_specs=[
            pl.BlockSpec((NBLK, 128), lambda i: (i, 0)),
            pl.BlockSpec((C, C), lambda i: (0, 0)),
            pl.BlockSpec((1, C), lambda i: (0, 0)),
            pl.BlockSpec((B, C), lambda i: (0, 0)),
        ],
        out_specs=[
            pl.BlockSpec((1, 1), lambda i: (0, 0)),
            pl.BlockSpec((C, C), lambda i: (0, 0)),
            pl.BlockSpec((1, C), lambda i: (0, 0)),
        ],
        out_shape=[
            jax.ShapeDtypeStruct((1, 1), jnp.float32),
            jax.ShapeDtypeStruct((C, C), jnp.float32),
            jax.ShapeDtypeStruct((1, C), jnp.float32),
        ],
    )(x3, arestT, astart, last)


def _sc_make():
    mesh = plsc.VectorSubcoreMesh(core_axis_name="c", subcore_axis_name="s")

    @functools.partial(
        pl.kernel,
        mesh=mesh,
        out_type=jax.ShapeDtypeStruct((NW, 16), jnp.float32),
        compiler_params=pltpu.CompilerParams(needs_layout_passes=False),
        scratch_types=[
            pltpu.VMEM((WCHUNK,), jnp.int32),
            pltpu.VMEM((WCHUNK,), jnp.int32),
            pltpu.VMEM((NROWIDX, 128), jnp.int32),
            pltpu.VMEM((NROWIDX, 128), jnp.float32),
            pltpu.VMEM((C * C,), jnp.float32),
            pltpu.VMEM((C,), jnp.float32),
            pltpu.VMEM((16,), jnp.int32),
            pltpu.VMEM((16,), jnp.float32),
            pltpu.SemaphoreType.DMA,
        ],
    )
    def sc(lab_hbm, nxt_hbm, lp_hbm, tab_hbm, astartn_hbm, lab0_hbm, out_hbm,
           lab_v, nxt_v, idx_v, gat_v, tab_v, astart_v, lab0_v, acc_v, sem):
        cid = lax.axis_index("c")
        sid = lax.axis_index("s")
        wid = sid * 2 + cid
        base = wid * WCHUNK

        pltpu.sync_copy(lab_hbm.at[pl.ds(base, WCHUNK)], lab_v)
        pltpu.sync_copy(nxt_hbm.at[pl.ds(base, WCHUNK)], nxt_v)
        pltpu.sync_copy(tab_hbm, tab_v)

        lane = lax.iota(jnp.int32, 16)

        def build(k, acc):
            l = lab_v[pl.ds(k * 16, 16)]
            nx = nxt_v[pl.ds(k * 16, 16)]
            idx = (base + k * 16) * C + lane * C + l
            idx_v[k // 8, pl.ds((k % 8) * 16, 16)] = idx
            tr = plsc.load_gather(tab_v, [nx * C + l])
            return acc + tr

        acc = lax.fori_loop(0, KV, build, jnp.zeros((16,), jnp.float32))

        def fire(j, _):
            pltpu.async_copy(lp_hbm.at[idx_v.at[j]], gat_v.at[j], sem)
            return 0
        lax.fori_loop(0, NROWIDX, fire, 0)

        def drain(j, _):
            pltpu.make_async_copy(lp_hbm.at[idx_v.at[j]], gat_v.at[j],
                                  sem).wait()
            return 0
        lax.fori_loop(0, NROWIDX, drain, 0)

        def esum(k, a):
            return a + gat_v[k // 8, pl.ds((k % 8) * 16, 16)]
        acc = lax.fori_loop(0, KV, esum, acc)

        @pl.when(wid == 0)
        def _start():
            pltpu.sync_copy(astartn_hbm, astart_v)
            pltpu.sync_copy(lab0_hbm, lab0_v)
            l0 = lab0_v[...]
            acc_v[...] = acc + plsc.load_gather(astart_v, [l0])

        @pl.when(wid != 0)
        def _nostart():
            acc_v[...] = acc

        pltpu.sync_copy(acc_v, out_hbm.at[wid])

    return sc


_sc_kernel = _sc_make()


def kernel(log_probs, input_lens, labels, A_scores):
    del input_lens
    x3 = log_probs.reshape(NLANE, 128)
    lp_flat = log_probs.reshape(ROWS * C)
    lab_flat = labels.reshape(ROWS)
    nxt_flat = jnp.concatenate(
        [labels[:, 1:], jnp.full((B, 1), L, dtype=labels.dtype)],
        axis=1).reshape(ROWS)
    arest_pad = jnp.concatenate(
        [A_scores[L:].reshape(L, C), jnp.zeros((1, C), jnp.float32)], axis=0)
    arestT = arest_pad.T
    astart = jnp.concatenate(
        [A_scores[:L], jnp.full((1,), -1e30, jnp.float32)]).reshape(1, C)
    lab0 = labels[:, 0]
    last = log_probs[:, -1, :]

    s_tc, tabT, astartn = _tc_call(x3, arestT, astart, last)
    sc_parts = _sc_kernel(lab_flat, nxt_flat, lp_flat,
                          tabT.reshape(C * C), astartn.reshape(C), lab0)
    return (s_tc[0, 0] + jnp.sum(sc_parts)) / float(ROWS)
